# baseline (speedup 1.0000x reference)
import jax
import jax.numpy as jnp
from jax import lax
from jax.experimental import pallas as pl
from jax.experimental.pallas import tpu as pltpu

N = 20000
K = 1024
C_CLS = 82
C_BOX = 324
BN = 2000
KB = 256

_DNUMS = (((1,), (1,)), ((), ()))


def _heads_kernel(x_ref, wc_ref, wb_ref, bc_ref, bb_ref, s_ref, d_ref):
    k = pl.program_id(1)
    xb = x_ref[...].astype(jnp.bfloat16)
    wc = wc_ref[...].astype(jnp.bfloat16)
    wb = wb_ref[...].astype(jnp.bfloat16)
    s = lax.dot_general(xb, wc, _DNUMS, preferred_element_type=jnp.float32)
    d = lax.dot_general(xb, wb, _DNUMS, preferred_element_type=jnp.float32)

    @pl.when(k == 0)
    def _init():
        s_ref[...] = s + bc_ref[...]
        d_ref[...] = d + bb_ref[...]

    @pl.when(k != 0)
    def _accum():
        s_ref[...] += s
        d_ref[...] += d


def kernel(x, W_cls, b_cls, W_bbox, b_bbox):
    if x.ndim > 2:
        x = x.reshape(x.shape[0], -1)
    bc = b_cls.reshape(1, C_CLS)
    bb = b_bbox.reshape(1, C_BOX)

    grid = (N // BN, K // KB)
    scores, deltas = pl.pallas_call(
        _heads_kernel,
        grid=grid,
        in_specs=[
            pl.BlockSpec((BN, KB), lambda i, k: (i, k)),
            pl.BlockSpec((C_CLS, KB), lambda i, k: (0, k)),
            pl.BlockSpec((C_BOX, KB), lambda i, k: (0, k)),
            pl.BlockSpec((1, C_CLS), lambda i, k: (0, 0)),
            pl.BlockSpec((1, C_BOX), lambda i, k: (0, 0)),
        ],
        out_specs=[
            pl.BlockSpec((BN, C_CLS), lambda i, k: (i, 0)),
            pl.BlockSpec((BN, C_BOX), lambda i, k: (i, 0)),
        ],
        out_shape=[
            jax.ShapeDtypeStruct((N, C_CLS), jnp.float32),
            jax.ShapeDtypeStruct((N, C_BOX), jnp.float32),
        ],
        compiler_params=pltpu.CompilerParams(
            dimension_semantics=("parallel", "arbitrary"),
        ),
    )(x, W_cls, W_bbox, bc, bb)
    return scores, deltas


# speedup vs baseline: 1.3818x; 1.3818x over previous
import jax
import jax.numpy as jnp
from jax import lax
from jax.experimental import pallas as pl
from jax.experimental.pallas import tpu as pltpu

N = 20000
K = 1024
C_CLS = 82
C_BOX = 324
BN = 2000

_DNUMS = (((1,), (1,)), ((), ()))


def _heads_kernel(x_ref, wc_ref, wb_ref, bc_ref, bb_ref, s_ref, d_ref):
    xb = x_ref[...].astype(jnp.bfloat16)
    wc = wc_ref[...].astype(jnp.bfloat16)
    wb = wb_ref[...].astype(jnp.bfloat16)
    s = lax.dot_general(xb, wc, _DNUMS, preferred_element_type=jnp.float32)
    d = lax.dot_general(xb, wb, _DNUMS, preferred_element_type=jnp.float32)
    s_ref[...] = s + bc_ref[...]
    d_ref[...] = d + bb_ref[...]


def kernel(x, W_cls, b_cls, W_bbox, b_bbox):
    if x.ndim > 2:
        x = x.reshape(x.shape[0], -1)
    bc = b_cls.reshape(1, C_CLS)
    bb = b_bbox.reshape(1, C_BOX)

    grid = (N // BN,)
    scores, deltas = pl.pallas_call(
        _heads_kernel,
        grid=grid,
        in_specs=[
            pl.BlockSpec((BN, K), lambda i: (i, 0)),
            pl.BlockSpec((C_CLS, K), lambda i: (0, 0)),
            pl.BlockSpec((C_BOX, K), lambda i: (0, 0)),
            pl.BlockSpec((1, C_CLS), lambda i: (0, 0)),
            pl.BlockSpec((1, C_BOX), lambda i: (0, 0)),
        ],
        out_specs=[
            pl.BlockSpec((BN, C_CLS), lambda i: (i, 0)),
            pl.BlockSpec((BN, C_BOX), lambda i: (i, 0)),
        ],
        out_shape=[
            jax.ShapeDtypeStruct((N, C_CLS), jnp.float32),
            jax.ShapeDtypeStruct((N, C_BOX), jnp.float32),
        ],
        compiler_params=pltpu.CompilerParams(
            dimension_semantics=("parallel",),
        ),
    )(x, W_cls, W_bbox, bc, bb)
    return scores, deltas
